# Initial kernel scaffold; baseline (speedup 1.0000x reference)
import functools

import jax
import jax.numpy as jnp
from jax import lax
from jax.experimental import pallas as pl
from jax.experimental.pallas import tpu as pltpu
from jax.experimental.pallas import tpu_sc as plsc

_N = 10000
_E = 320000
_D = 128
_NPAD = 10240
_NCORES = 2
_NSUB = 16
_NW = _NCORES * _NSUB
_CH = 128
_NCH = 80
_EPAD = _NW * _NCH * _CH
_ZROWS = _NPAD // _NSUB


def _mm_body(x_ref, w_ref, o_ref):
    o_ref[...] = jnp.dot(x_ref[...], w_ref[...],
                         preferred_element_type=jnp.float32,
                         precision=lax.Precision.HIGHEST)


_matmul1 = pl.pallas_call(
    _mm_body,
    grid=(5,),
    in_specs=[pl.BlockSpec((2000, _D), lambda i: (i, 0)),
              pl.BlockSpec((_D, _D), lambda i: (0, 0))],
    out_specs=pl.BlockSpec((2000, _D), lambda i: (i, 0)),
    out_shape=jax.ShapeDtypeStruct((_N, _D), jnp.float32),
)


def _mm2_body(a_ref, b_ref, w_ref, o_ref):
    h = jnp.maximum(a_ref[...] + b_ref[...], 0.0)
    o_ref[...] = jnp.dot(h, w_ref[...],
                         preferred_element_type=jnp.float32,
                         precision=lax.Precision.HIGHEST)


_matmul2 = pl.pallas_call(
    _mm2_body,
    grid=(8,),
    in_specs=[pl.BlockSpec((1280, _D), lambda i: (i, 0)),
              pl.BlockSpec((1280, _D), lambda i: (i, 0)),
              pl.BlockSpec((_D, _D), lambda i: (0, 0))],
    out_specs=pl.BlockSpec((1280, _D), lambda i: (i, 0)),
    out_shape=jax.ShapeDtypeStruct((_NPAD, _D), jnp.float32),
)


def _lsm_body(a_ref, b_ref, o_ref):
    h = a_ref[...] + b_ref[...]
    m = jnp.max(h, axis=-1, keepdims=True)
    e = jnp.exp(h - m)
    s = jnp.sum(e, axis=-1, keepdims=True)
    o_ref[...] = h - m - jnp.log(s)


_logsoftmax = pl.pallas_call(
    _lsm_body,
    grid=(5,),
    in_specs=[pl.BlockSpec((2000, _D), lambda i: (i, 0)),
              pl.BlockSpec((2000, _D), lambda i: (i, 0))],
    out_specs=pl.BlockSpec((2000, _D), lambda i: (i, 0)),
    out_shape=jax.ShapeDtypeStruct((_N, _D), jnp.float32),
)


def _make_agg(h_rows):
    del h_rows
    mesh = plsc.VectorSubcoreMesh(core_axis_name="c", subcore_axis_name="s")

    @functools.partial(
        pl.kernel,
        mesh=mesh,
        out_type=(jax.ShapeDtypeStruct((_NPAD, _D), jnp.float32),
                  jax.ShapeDtypeStruct((_NPAD, _D), jnp.float32)),
        scratch_types=[
            pltpu.VMEM((_NCH, _CH), jnp.int32),
            pltpu.VMEM((_NCH, _CH), jnp.int32),
            pltpu.VMEM((_CH, _D), jnp.float32),
            pltpu.VMEM((_CH, _D), jnp.float32),
            pltpu.VMEM_SHARED((_NPAD, _D), jnp.float32),
            pltpu.SemaphoreType.DMA,
            pltpu.SemaphoreType.DMA,
        ],
    )
    def agg(h_hbm, src_hbm, dst_hbm, zeros_hbm, out0_hbm, out1_hbm,
            src_v, dst_v, buf0, buf1, acc_sh, sem0, sem1):
        c = lax.axis_index("c")
        s = lax.axis_index("s")
        base = (c * _NSUB + s) * _NCH
        zbase = s * _ZROWS

        pltpu.sync_copy(src_hbm.at[pl.ds(base, _NCH)], src_v)
        pltpu.sync_copy(dst_hbm.at[pl.ds(base, _NCH)], dst_v)
        pltpu.sync_copy(zeros_hbm, acc_sh.at[pl.ds(zbase, _ZROWS)])
        plsc.subcore_barrier()

        pltpu.async_copy(h_hbm.at[src_v.at[0]], buf0, sem0)

        @pl.loop(0, _NCH, step=2)
        def _edges(i):
            pltpu.async_copy(h_hbm.at[src_v.at[i + 1]], buf1, sem1)
            pltpu.make_async_copy(h_hbm.at[src_v.at[i]], buf0, sem0).wait()
            pltpu.sync_copy(buf0, acc_sh.at[dst_v.at[i]], add=True)

            @pl.when(i + 2 < _NCH)
            def _():
                pltpu.async_copy(h_hbm.at[src_v.at[i + 2]], buf0, sem0)

            pltpu.make_async_copy(h_hbm.at[src_v.at[i + 1]], buf1, sem1).wait()
            pltpu.sync_copy(buf1, acc_sh.at[dst_v.at[i + 1]], add=True)

        plsc.subcore_barrier()

        @pl.when(c == 0)
        def _():
            pltpu.sync_copy(acc_sh.at[pl.ds(zbase, _ZROWS)],
                            out0_hbm.at[pl.ds(zbase, _ZROWS)])

        @pl.when(c == 1)
        def _():
            pltpu.sync_copy(acc_sh.at[pl.ds(zbase, _ZROWS)],
                            out1_hbm.at[pl.ds(zbase, _ZROWS)])

    return agg


_agg1 = _make_agg(_N)
_agg2 = _make_agg(_NPAD)


def kernel(x, edge_index, W1, W2):
    src = edge_index[0].astype(jnp.int32)
    dst = edge_index[1].astype(jnp.int32)
    npad = _EPAD - _E
    src2d = jnp.concatenate(
        [src, jnp.zeros((npad,), jnp.int32)]).reshape(-1, _CH)
    dst2d = jnp.concatenate(
        [dst, jnp.full((npad,), _N, jnp.int32)]).reshape(-1, _CH)
    zeros = jnp.zeros((_ZROWS, _D), jnp.float32)

    h1 = _matmul1(x, W1.T)
    p0, p1 = _agg1(h1, src2d, dst2d, zeros)
    h2 = _matmul2(p0, p1, W2.T)
    q0, q1 = _agg2(h2, src2d, dst2d, zeros)
    return _logsoftmax(q0, q1)


# speedup vs baseline: 4.8634x; 4.8634x over previous
import functools

import jax
import jax.numpy as jnp
from jax import lax
from jax.experimental import pallas as pl
from jax.experimental.pallas import tpu as pltpu
from jax.experimental.pallas import tpu_sc as plsc

_N = 10000
_E = 320000
_D = 128
_DH = 64
_NPAD = 10240
_NSUB = 16
_CH = 128
_NCH = 160
_EPAD = _NSUB * _NCH * _CH
_ZROWS = _NPAD // _NSUB


def _mm1_body(x_ref, w_ref, oa_ref, ob_ref):
    h = jnp.dot(x_ref[...], w_ref[...],
                preferred_element_type=jnp.float32,
                precision=lax.Precision.HIGHEST)
    oa_ref[...] = h[:, :_DH]
    ob_ref[...] = h[:, _DH:]


_matmul1 = pl.pallas_call(
    _mm1_body,
    grid=(5,),
    in_specs=[pl.BlockSpec((2000, _D), lambda i: (i, 0)),
              pl.BlockSpec((_D, _D), lambda i: (0, 0))],
    out_specs=[pl.BlockSpec((2000, _DH), lambda i: (i, 0)),
               pl.BlockSpec((2000, _DH), lambda i: (i, 0))],
    out_shape=(jax.ShapeDtypeStruct((_N, _DH), jnp.float32),
               jax.ShapeDtypeStruct((_N, _DH), jnp.float32)),
)


def _mm2_body(a_ref, b_ref, w_ref, oa_ref, ob_ref):
    h = jnp.maximum(jnp.concatenate([a_ref[...], b_ref[...]], axis=-1), 0.0)
    h = jnp.dot(h, w_ref[...],
                preferred_element_type=jnp.float32,
                precision=lax.Precision.HIGHEST)
    oa_ref[...] = h[:, :_DH]
    ob_ref[...] = h[:, _DH:]


_matmul2 = pl.pallas_call(
    _mm2_body,
    grid=(8,),
    in_specs=[pl.BlockSpec((1280, _DH), lambda i: (i, 0)),
              pl.BlockSpec((1280, _DH), lambda i: (i, 0)),
              pl.BlockSpec((_D, _D), lambda i: (0, 0))],
    out_specs=[pl.BlockSpec((1280, _DH), lambda i: (i, 0)),
               pl.BlockSpec((1280, _DH), lambda i: (i, 0))],
    out_shape=(jax.ShapeDtypeStruct((_NPAD, _DH), jnp.float32),
               jax.ShapeDtypeStruct((_NPAD, _DH), jnp.float32)),
)


def _lsm_body(a_ref, b_ref, o_ref):
    h = jnp.concatenate([a_ref[...], b_ref[...]], axis=-1)
    m = jnp.max(h, axis=-1, keepdims=True)
    e = jnp.exp(h - m)
    s = jnp.sum(e, axis=-1, keepdims=True)
    o_ref[...] = h - m - jnp.log(s)


_logsoftmax = pl.pallas_call(
    _lsm_body,
    grid=(5,),
    in_specs=[pl.BlockSpec((2000, _DH), lambda i: (i, 0)),
              pl.BlockSpec((2000, _DH), lambda i: (i, 0))],
    out_specs=pl.BlockSpec((2000, _D), lambda i: (i, 0)),
    out_shape=jax.ShapeDtypeStruct((_N, _D), jnp.float32),
)


def _make_agg():
    mesh = plsc.VectorSubcoreMesh(core_axis_name="c", subcore_axis_name="s")

    @functools.partial(
        pl.kernel,
        mesh=mesh,
        compiler_params=pltpu.CompilerParams(use_tc_tiling_on_sc=False),
        out_type=(jax.ShapeDtypeStruct((_NPAD, _DH), jnp.float32),
                  jax.ShapeDtypeStruct((_NPAD, _DH), jnp.float32)),
        scratch_types=[
            pltpu.VMEM((_NCH, _CH), jnp.int32),
            pltpu.VMEM((_NCH, _CH), jnp.int32),
            pltpu.VMEM((_CH, _DH), jnp.float32),
            pltpu.VMEM((_CH, _DH), jnp.float32),
            pltpu.VMEM_SHARED((_NPAD, _DH), jnp.float32),
            pltpu.SemaphoreType.DMA,
            pltpu.SemaphoreType.DMA,
        ],
    )
    def agg(ha_hbm, hb_hbm, src_hbm, dst_hbm, zeros_hbm, outa_hbm, outb_hbm,
            src_v, dst_v, buf0, buf1, acc_sh, sem0, sem1):
        c = lax.axis_index("c")
        s = lax.axis_index("s")
        base = s * _NCH
        zbase = s * _ZROWS

        pltpu.sync_copy(src_hbm.at[pl.ds(base, _NCH)], src_v)
        pltpu.sync_copy(dst_hbm.at[pl.ds(base, _NCH)], dst_v)
        pltpu.sync_copy(zeros_hbm, acc_sh.at[pl.ds(zbase, _ZROWS)])
        plsc.subcore_barrier()

        def edge_loop(h_hbm):
            pltpu.async_copy(h_hbm.at[src_v.at[0]], buf0, sem0)

            @pl.loop(0, _NCH, step=2)
            def _edges(i):
                pltpu.async_copy(h_hbm.at[src_v.at[i + 1]], buf1, sem1)
                pltpu.make_async_copy(h_hbm.at[src_v.at[i]], buf0, sem0).wait()
                pltpu.sync_copy(buf0, acc_sh.at[dst_v.at[i]], add=True)

                @pl.when(i + 2 < _NCH)
                def _():
                    pltpu.async_copy(h_hbm.at[src_v.at[i + 2]], buf0, sem0)

                pltpu.make_async_copy(
                    h_hbm.at[src_v.at[i + 1]], buf1, sem1).wait()
                pltpu.sync_copy(buf1, acc_sh.at[dst_v.at[i + 1]], add=True)

        @pl.when(c == 0)
        def _():
            edge_loop(ha_hbm)

        @pl.when(c == 1)
        def _():
            edge_loop(hb_hbm)

        plsc.subcore_barrier()

        @pl.when(c == 0)
        def _():
            pltpu.sync_copy(acc_sh.at[pl.ds(zbase, _ZROWS)],
                            outa_hbm.at[pl.ds(zbase, _ZROWS)])

        @pl.when(c == 1)
        def _():
            pltpu.sync_copy(acc_sh.at[pl.ds(zbase, _ZROWS)],
                            outb_hbm.at[pl.ds(zbase, _ZROWS)])

    return agg


_agg = _make_agg()


def kernel(x, edge_index, W1, W2):
    src = edge_index[0].astype(jnp.int32)
    dst = edge_index[1].astype(jnp.int32)
    npad = _EPAD - _E
    src2d = jnp.concatenate(
        [src, jnp.zeros((npad,), jnp.int32)]).reshape(-1, _CH)
    dst2d = jnp.concatenate(
        [dst, jnp.full((npad,), _N, jnp.int32)]).reshape(-1, _CH)
    zeros = jnp.zeros((_ZROWS, _DH), jnp.float32)

    ha, hb = _matmul1(x, W1.T)
    pa, pb = _agg(ha, hb, src2d, dst2d, zeros)
    ga, gb = _matmul2(pa, pb, W2.T)
    qa, qb = _agg(ga, gb, src2d, dst2d, zeros)
    return _logsoftmax(qa, qb)
